# baseline (speedup 1.0000x reference)
import functools

import jax
import jax.numpy as jnp
from jax import lax
from jax.experimental import pallas as pl
from jax.experimental.pallas import tpu as pltpu
from jax.experimental.pallas import tpu_sc as plsc

DIM = 64
WIDE = 128
LANES = 16
NC = 2
NS = 16
NW = NC * NS
TOTAL = 32768
PER_W = TOTAL // NW
CHB = 128
NCH = PER_W // CHB

EPS2 = 1e-24
TINY = 1e-35


def _fast_rsqrt(x):
    i = lax.bitcast_convert_type(x, jnp.int32)
    i = jnp.int32(0x5F3759DF) - lax.shift_right_arithmetic(i, 1)
    y = lax.bitcast_convert_type(i, jnp.float32)
    half_x = 0.5 * x
    for _ in range(3):
        y = y * (1.5 - half_x * y * y)
    return y


def _transe_body(ents_hbm, rels_hbm, hidx_hbm, ridx_hbm, tidx_hbm, out_hbm,
                 hidx_v, ridx_v, tidx_v, hhalf, rhalf, thalf,
                 hoff, roff, toff, h0, r0, t0, h1, r1, t1,
                 scores_v, sem0, sem1):
    wid = lax.axis_index("s") * NC + lax.axis_index("c")
    iota = lax.iota(jnp.int32, LANES)
    base = wid * PER_W

    pltpu.sync_copy(hidx_hbm.at[pl.ds(base, PER_W)], hidx_v)
    pltpu.sync_copy(ridx_hbm.at[pl.ds(base, PER_W)], ridx_v)
    pltpu.sync_copy(tidx_hbm.at[pl.ds(base, PER_W)], tidx_v)

    def split_body(j, _):
        s = pl.ds(j * LANES, LANES)
        for idx_v, half_v, off_v in ((hidx_v, hhalf, hoff),
                                     (ridx_v, rhalf, roff),
                                     (tidx_v, thalf, toff)):
            v = idx_v[s]
            half_v[s] = lax.shift_left(lax.shift_right_logical(v, 12), 11) \
                + jnp.bitwise_and(v, 2047)
            off_v[s] = jnp.bitwise_and(lax.shift_right_logical(v, 11), 1)
        return 0

    lax.fori_loop(0, PER_W // LANES, split_body, 0)

    def fire(c, hb, rb, tb, sem):
        cc = jnp.minimum(c, NCH - 1)
        sl = pl.ds(cc * CHB, CHB)
        pltpu.async_copy(ents_hbm.at[hhalf.at[sl]], hb, sem)
        pltpu.async_copy(rels_hbm.at[rhalf.at[sl]], rb, sem)
        pltpu.async_copy(ents_hbm.at[thalf.at[sl]], tb, sem)

    def drain(hb, rb, tb, sem):
        pltpu.make_async_copy(ents_hbm.at[pl.ds(0, CHB)], hb, sem).wait()
        pltpu.make_async_copy(rels_hbm.at[pl.ds(0, CHB)], rb, sem).wait()
        pltpu.make_async_copy(ents_hbm.at[pl.ds(0, CHB)], tb, sem).wait()

    def compute(c, hb, rb, tb):
        def o_body(oi, _):
            s = pl.ds(c * CHB + oi * LANES, LANES)
            rows16 = oi * LANES + iota
            ho = hoff[s]
            ro = roff[s]
            to = toff[s]
            ssq = jnp.zeros((LANES,), jnp.float32)
            for d in range(DIM):
                vh = plsc.load_gather(hb, [rows16, ho + 2 * d])
                vr = plsc.load_gather(rb, [rows16, ro + 2 * d])
                vt = plsc.load_gather(tb, [rows16, to + 2 * d])
                diff = vh + vr - vt
                ssq = ssq + diff * diff
            scores_v[s] = ssq * _fast_rsqrt(jnp.maximum(ssq, TINY))
            return 0

        lax.fori_loop(0, CHB // LANES, o_body, 0)

    fire(jnp.int32(0), h0, r0, t0, sem0)

    def pair_body(i, _):
        c0 = i * 2
        drain(h0, r0, t0, sem0)
        fire(c0 + 1, h1, r1, t1, sem1)
        compute(c0, h0, r0, t0)
        drain(h1, r1, t1, sem1)
        fire(c0 + 2, h0, r0, t0, sem0)
        compute(c0 + 1, h1, r1, t1)
        return 0

    lax.fori_loop(0, NCH // 2, pair_body, 0)
    drain(h0, r0, t0, sem0)
    pltpu.sync_copy(scores_v, out_hbm.at[pl.ds(base, PER_W)])


@functools.partial(
    pl.kernel,
    out_type=jax.ShapeDtypeStruct((TOTAL,), jnp.float32),
    mesh=plsc.VectorSubcoreMesh(core_axis_name="c", subcore_axis_name="s"),
    compiler_params=pltpu.CompilerParams(
        needs_layout_passes=False, use_tc_tiling_on_sc=True),
    scratch_types=[
        pltpu.VMEM((PER_W,), jnp.int32),
        pltpu.VMEM((PER_W,), jnp.int32),
        pltpu.VMEM((PER_W,), jnp.int32),
        pltpu.VMEM((PER_W,), jnp.int32),
        pltpu.VMEM((PER_W,), jnp.int32),
        pltpu.VMEM((PER_W,), jnp.int32),
        pltpu.VMEM((PER_W,), jnp.int32),
        pltpu.VMEM((PER_W,), jnp.int32),
        pltpu.VMEM((PER_W,), jnp.int32),
        pltpu.VMEM((CHB, WIDE), jnp.float32),
        pltpu.VMEM((CHB, WIDE), jnp.float32),
        pltpu.VMEM((CHB, WIDE), jnp.float32),
        pltpu.VMEM((CHB, WIDE), jnp.float32),
        pltpu.VMEM((CHB, WIDE), jnp.float32),
        pltpu.VMEM((CHB, WIDE), jnp.float32),
        pltpu.VMEM((PER_W,), jnp.float32),
        pltpu.SemaphoreType.DMA,
        pltpu.SemaphoreType.DMA,
    ],
)
def _transe_sc(*refs):
    _transe_body(*refs)


TBLK = 4096
HALF_T = TBLK // 2


def _xpose_body(x_ref, y_ref):
    x = x_ref[...]
    norm = jnp.sqrt(jnp.sum(x * x, axis=0, keepdims=True))
    xn = x / jnp.maximum(norm, 1e-12)
    y_ref[...] = xn.reshape(WIDE, HALF_T).T


def _rows_table(table_t, nblk):
    return pl.pallas_call(
        _xpose_body,
        grid=(nblk,),
        in_specs=[pl.BlockSpec((DIM, TBLK), lambda j: (0, j))],
        out_specs=pl.BlockSpec((HALF_T, WIDE), lambda j: (j, 0)),
        out_shape=jax.ShapeDtypeStruct((nblk * HALF_T, WIDE), jnp.float32),
    )(table_t)


def kernel(ents_w, rels_w, heads, rels, tails, sources, heads_bad, rels_bad,
           tails_bad, sources_bad):
    b = heads.shape[0]
    n_ents = ents_w.shape[0]
    nblk = (n_ents + TBLK - 1) // TBLK
    ents_rows = _rows_table(ents_w.T, nblk)
    rels_rows = _rows_table(rels_w.T, 1)
    all_heads = jnp.concatenate([heads, heads_bad])
    all_rels = jnp.concatenate([rels, rels_bad])
    all_tails = jnp.concatenate([tails, tails_bad])
    out = _transe_sc(ents_rows, rels_rows, all_heads, all_rels, all_tails)
    return out[:b], out[b:]


# speedup vs baseline: 1.2197x; 1.2197x over previous
import functools

import jax
import jax.numpy as jnp
from jax import lax
from jax.experimental import pallas as pl
from jax.experimental.pallas import tpu as pltpu
from jax.experimental.pallas import tpu_sc as plsc

DIM = 64
WIDE = 128
LANES = 16
NC = 2
NS = 16
NW = NC * NS
TOTAL = 32768
PER_W = TOTAL // NW
CHB = 128
NCH = PER_W // CHB

EPS2 = 1e-24
TINY = 1e-35


def _fast_rsqrt(x):
    i = lax.bitcast_convert_type(x, jnp.int32)
    i = jnp.int32(0x5F3759DF) - lax.shift_right_arithmetic(i, 1)
    y = lax.bitcast_convert_type(i, jnp.float32)
    half_x = 0.5 * x
    for _ in range(3):
        y = y * (1.5 - half_x * y * y)
    return y


def _transe_body(ents_hbm, rels_hbm, hidx_hbm, ridx_hbm, tidx_hbm, out_hbm,
                 hidx_v, ridx_v, tidx_v, hhalf, rhalf, thalf,
                 hoff, roff, toff, h0, r0, t0, h1, r1, t1,
                 scores_v, sem0, sem1):
    wid = lax.axis_index("s") * NC + lax.axis_index("c")
    iota = lax.iota(jnp.int32, LANES)
    base = wid * PER_W

    pltpu.sync_copy(hidx_hbm.at[pl.ds(base, PER_W)], hidx_v)
    pltpu.sync_copy(ridx_hbm.at[pl.ds(base, PER_W)], ridx_v)
    pltpu.sync_copy(tidx_hbm.at[pl.ds(base, PER_W)], tidx_v)

    def split_body(j, _):
        s = pl.ds(j * LANES, LANES)
        for idx_v, half_v, off_v in ((hidx_v, hhalf, hoff),
                                     (ridx_v, rhalf, roff),
                                     (tidx_v, thalf, toff)):
            v = idx_v[s]
            half_v[s] = lax.shift_left(lax.shift_right_logical(v, 13), 12) \
                + jnp.bitwise_and(v, 4095)
            off_v[s] = jnp.bitwise_and(lax.shift_right_logical(v, 12), 1)
        return 0

    lax.fori_loop(0, PER_W // LANES, split_body, 0)

    def fire(c, hb, rb, tb, sem):
        cc = jnp.minimum(c, NCH - 1)
        sl = pl.ds(cc * CHB, CHB)
        pltpu.async_copy(ents_hbm.at[hhalf.at[sl]], hb, sem)
        pltpu.async_copy(rels_hbm.at[rhalf.at[sl]], rb, sem)
        pltpu.async_copy(ents_hbm.at[thalf.at[sl]], tb, sem)

    def drain(hb, rb, tb, sem):
        pltpu.make_async_copy(ents_hbm.at[pl.ds(0, CHB)], hb, sem).wait()
        pltpu.make_async_copy(rels_hbm.at[pl.ds(0, CHB)], rb, sem).wait()
        pltpu.make_async_copy(ents_hbm.at[pl.ds(0, CHB)], tb, sem).wait()

    def compute(c, hb, rb, tb):
        def o_body(oi, _):
            s = pl.ds(c * CHB + oi * LANES, LANES)
            rows16 = oi * LANES + iota
            ho = hoff[s]
            ro = roff[s]
            to = toff[s]
            ssq = jnp.zeros((LANES,), jnp.float32)
            for d in range(DIM):
                vh = plsc.load_gather(hb, [rows16, ho + 2 * d])
                vr = plsc.load_gather(rb, [rows16, ro + 2 * d])
                vt = plsc.load_gather(tb, [rows16, to + 2 * d])
                diff = vh + vr - vt
                ssq = ssq + diff * diff
            scores_v[s] = ssq * _fast_rsqrt(jnp.maximum(ssq, TINY))
            return 0

        lax.fori_loop(0, CHB // LANES, o_body, 0)

    fire(jnp.int32(0), h0, r0, t0, sem0)

    def pair_body(i, _):
        c0 = i * 2
        drain(h0, r0, t0, sem0)
        fire(c0 + 1, h1, r1, t1, sem1)
        compute(c0, h0, r0, t0)
        drain(h1, r1, t1, sem1)
        fire(c0 + 2, h0, r0, t0, sem0)
        compute(c0 + 1, h1, r1, t1)
        return 0

    lax.fori_loop(0, NCH // 2, pair_body, 0)
    drain(h0, r0, t0, sem0)
    pltpu.sync_copy(scores_v, out_hbm.at[pl.ds(base, PER_W)])


@functools.partial(
    pl.kernel,
    out_type=jax.ShapeDtypeStruct((TOTAL,), jnp.float32),
    mesh=plsc.VectorSubcoreMesh(core_axis_name="c", subcore_axis_name="s"),
    compiler_params=pltpu.CompilerParams(
        needs_layout_passes=False, use_tc_tiling_on_sc=True),
    scratch_types=[
        pltpu.VMEM((PER_W,), jnp.int32),
        pltpu.VMEM((PER_W,), jnp.int32),
        pltpu.VMEM((PER_W,), jnp.int32),
        pltpu.VMEM((PER_W,), jnp.int32),
        pltpu.VMEM((PER_W,), jnp.int32),
        pltpu.VMEM((PER_W,), jnp.int32),
        pltpu.VMEM((PER_W,), jnp.int32),
        pltpu.VMEM((PER_W,), jnp.int32),
        pltpu.VMEM((PER_W,), jnp.int32),
        pltpu.VMEM((CHB, WIDE), jnp.float32),
        pltpu.VMEM((CHB, WIDE), jnp.float32),
        pltpu.VMEM((CHB, WIDE), jnp.float32),
        pltpu.VMEM((CHB, WIDE), jnp.float32),
        pltpu.VMEM((CHB, WIDE), jnp.float32),
        pltpu.VMEM((CHB, WIDE), jnp.float32),
        pltpu.VMEM((PER_W,), jnp.float32),
        pltpu.SemaphoreType.DMA,
        pltpu.SemaphoreType.DMA,
    ],
)
def _transe_sc(*refs):
    _transe_body(*refs)


TBLK = 8192
HALF_T = TBLK // 2


def _xpose_body(x_ref, y_ref):
    x = x_ref[...]
    norm = jnp.sqrt(jnp.sum(x * x, axis=0, keepdims=True))
    xn = x / jnp.maximum(norm, 1e-12)
    y_ref[...] = xn.reshape(WIDE, HALF_T).T


def _rows_table(table_t, nblk):
    return pl.pallas_call(
        _xpose_body,
        grid=(nblk,),
        in_specs=[pl.BlockSpec((DIM, TBLK), lambda j: (0, j))],
        out_specs=pl.BlockSpec((HALF_T, WIDE), lambda j: (j, 0)),
        out_shape=jax.ShapeDtypeStruct((nblk * HALF_T, WIDE), jnp.float32),
    )(table_t)


def kernel(ents_w, rels_w, heads, rels, tails, sources, heads_bad, rels_bad,
           tails_bad, sources_bad):
    b = heads.shape[0]
    n_ents = ents_w.shape[0]
    nblk = (n_ents + TBLK - 1) // TBLK
    ents_rows = _rows_table(ents_w.T, nblk)
    rels_rows = _rows_table(rels_w.T, 1)
    all_heads = jnp.concatenate([heads, heads_bad])
    all_rels = jnp.concatenate([rels, rels_bad])
    all_tails = jnp.concatenate([tails, tails_bad])
    out = _transe_sc(ents_rows, rels_rows, all_heads, all_rels, all_tails)
    return out[:b], out[b:]


# speedup vs baseline: 1.3620x; 1.1166x over previous
import functools

import jax
import jax.numpy as jnp
from jax import lax
from jax.experimental import pallas as pl
from jax.experimental.pallas import tpu as pltpu
from jax.experimental.pallas import tpu_sc as plsc

DIM = 64
WIDE = 128
LANES = 16
NC = 2
NS = 16
NW = NC * NS
TOTAL = 32768
PER_W = TOTAL // NW
CHB = 128
NCH = PER_W // CHB

EPS2 = 1e-24
TINY = 1e-35


def _fast_rsqrt(x):
    i = lax.bitcast_convert_type(x, jnp.int32)
    i = jnp.int32(0x5F3759DF) - lax.shift_right_arithmetic(i, 1)
    y = lax.bitcast_convert_type(i, jnp.float32)
    half_x = 0.5 * x
    for _ in range(3):
        y = y * (1.5 - half_x * y * y)
    return y


def _transe_body(ents_hbm, rels_hbm, hidx_hbm, ridx_hbm, tidx_hbm, out_hbm,
                 hidx_v, ridx_v, tidx_v, hhalf, rhalf, thalf,
                 hoff, roff, toff, h0, r0, t0, h1, r1, t1,
                 scores_v, sem0, sem1):
    wid = lax.axis_index("s") * NC + lax.axis_index("c")
    iota = lax.iota(jnp.int32, LANES)
    base = wid * PER_W

    pltpu.sync_copy(hidx_hbm.at[pl.ds(base, PER_W)], hidx_v)
    pltpu.sync_copy(ridx_hbm.at[pl.ds(base, PER_W)], ridx_v)
    pltpu.sync_copy(tidx_hbm.at[pl.ds(base, PER_W)], tidx_v)

    def split_body(j, _):
        s = pl.ds(j * LANES, LANES)
        for idx_v, half_v, off_v in ((hidx_v, hhalf, hoff),
                                     (ridx_v, rhalf, roff),
                                     (tidx_v, thalf, toff)):
            v = idx_v[s]
            half_v[s] = lax.shift_left(lax.shift_right_logical(v, 14), 13) \
                + jnp.bitwise_and(v, 8191)
            off_v[s] = jnp.bitwise_and(lax.shift_right_logical(v, 13), 1)
        return 0

    lax.fori_loop(0, PER_W // LANES, split_body, 0)

    def fire(c, hb, rb, tb, sem):
        cc = jnp.minimum(c, NCH - 1)
        sl = pl.ds(cc * CHB, CHB)
        pltpu.async_copy(ents_hbm.at[hhalf.at[sl]], hb, sem)
        pltpu.async_copy(rels_hbm.at[rhalf.at[sl]], rb, sem)
        pltpu.async_copy(ents_hbm.at[thalf.at[sl]], tb, sem)

    def drain(hb, rb, tb, sem):
        pltpu.make_async_copy(ents_hbm.at[pl.ds(0, CHB)], hb, sem).wait()
        pltpu.make_async_copy(rels_hbm.at[pl.ds(0, CHB)], rb, sem).wait()
        pltpu.make_async_copy(ents_hbm.at[pl.ds(0, CHB)], tb, sem).wait()

    def compute(c, hb, rb, tb):
        def o_body(oi, _):
            s = pl.ds(c * CHB + oi * LANES, LANES)
            rows16 = oi * LANES + iota
            ho = hoff[s]
            ro = roff[s]
            to = toff[s]
            ssq = jnp.zeros((LANES,), jnp.float32)
            for d in range(DIM):
                vh = plsc.load_gather(hb, [rows16, ho + 2 * d])
                vr = plsc.load_gather(rb, [rows16, ro + 2 * d])
                vt = plsc.load_gather(tb, [rows16, to + 2 * d])
                diff = vh + vr - vt
                ssq = ssq + diff * diff
            scores_v[s] = ssq * _fast_rsqrt(jnp.maximum(ssq, TINY))
            return 0

        lax.fori_loop(0, CHB // LANES, o_body, 0)

    fire(jnp.int32(0), h0, r0, t0, sem0)

    def pair_body(i, _):
        c0 = i * 2
        drain(h0, r0, t0, sem0)
        fire(c0 + 1, h1, r1, t1, sem1)
        compute(c0, h0, r0, t0)
        drain(h1, r1, t1, sem1)
        fire(c0 + 2, h0, r0, t0, sem0)
        compute(c0 + 1, h1, r1, t1)
        return 0

    lax.fori_loop(0, NCH // 2, pair_body, 0)
    drain(h0, r0, t0, sem0)
    pltpu.sync_copy(scores_v, out_hbm.at[pl.ds(base, PER_W)])


@functools.partial(
    pl.kernel,
    out_type=jax.ShapeDtypeStruct((TOTAL,), jnp.float32),
    mesh=plsc.VectorSubcoreMesh(core_axis_name="c", subcore_axis_name="s"),
    compiler_params=pltpu.CompilerParams(
        needs_layout_passes=False, use_tc_tiling_on_sc=True),
    scratch_types=[
        pltpu.VMEM((PER_W,), jnp.int32),
        pltpu.VMEM((PER_W,), jnp.int32),
        pltpu.VMEM((PER_W,), jnp.int32),
        pltpu.VMEM((PER_W,), jnp.int32),
        pltpu.VMEM((PER_W,), jnp.int32),
        pltpu.VMEM((PER_W,), jnp.int32),
        pltpu.VMEM((PER_W,), jnp.int32),
        pltpu.VMEM((PER_W,), jnp.int32),
        pltpu.VMEM((PER_W,), jnp.int32),
        pltpu.VMEM((CHB, WIDE), jnp.float32),
        pltpu.VMEM((CHB, WIDE), jnp.float32),
        pltpu.VMEM((CHB, WIDE), jnp.float32),
        pltpu.VMEM((CHB, WIDE), jnp.float32),
        pltpu.VMEM((CHB, WIDE), jnp.float32),
        pltpu.VMEM((CHB, WIDE), jnp.float32),
        pltpu.VMEM((PER_W,), jnp.float32),
        pltpu.SemaphoreType.DMA,
        pltpu.SemaphoreType.DMA,
    ],
)
def _transe_sc(*refs):
    _transe_body(*refs)


TBLK = 16384
HALF_T = TBLK // 2


def _xpose_body(x_ref, y_ref):
    x = x_ref[...]
    norm = jnp.sqrt(jnp.sum(x * x, axis=0, keepdims=True))
    xn = x / jnp.maximum(norm, 1e-12)
    y_ref[...] = xn.reshape(WIDE, HALF_T).T


def _rows_table(table_t, nblk):
    return pl.pallas_call(
        _xpose_body,
        grid=(nblk,),
        in_specs=[pl.BlockSpec((DIM, TBLK), lambda j: (0, j))],
        out_specs=pl.BlockSpec((HALF_T, WIDE), lambda j: (j, 0)),
        out_shape=jax.ShapeDtypeStruct((nblk * HALF_T, WIDE), jnp.float32),
    )(table_t)


def kernel(ents_w, rels_w, heads, rels, tails, sources, heads_bad, rels_bad,
           tails_bad, sources_bad):
    b = heads.shape[0]
    n_ents = ents_w.shape[0]
    nblk = (n_ents + TBLK - 1) // TBLK
    ents_rows = _rows_table(ents_w.T, nblk)
    rels_rows = _rows_table(rels_w.T, 1)
    all_heads = jnp.concatenate([heads, heads_bad])
    all_rels = jnp.concatenate([rels, rels_bad])
    all_tails = jnp.concatenate([tails, tails_bad])
    out = _transe_sc(ents_rows, rels_rows, all_heads, all_rels, all_tails)
    return out[:b], out[b:]


# speedup vs baseline: 1.4362x; 1.0544x over previous
import functools

import jax
import jax.numpy as jnp
from jax import lax
from jax.experimental import pallas as pl
from jax.experimental.pallas import tpu as pltpu
from jax.experimental.pallas import tpu_sc as plsc

DIM = 64
WIDE = 128
LANES = 16
NC = 2
NS = 16
NW = NC * NS
TOTAL = 32768
PER_W = TOTAL // NW
CHB = 128
NCH = PER_W // CHB

EPS2 = 1e-24
TINY = 1e-35


def _fast_rsqrt(x):
    i = lax.bitcast_convert_type(x, jnp.int32)
    i = jnp.int32(0x5F3759DF) - lax.shift_right_arithmetic(i, 1)
    y = lax.bitcast_convert_type(i, jnp.float32)
    half_x = 0.5 * x
    for _ in range(3):
        y = y * (1.5 - half_x * y * y)
    return y


def _transe_body(ents_hbm, rels_hbm, hidx_hbm, ridx_hbm, tidx_hbm, out_hbm,
                 hidx_v, ridx_v, tidx_v, hhalf, rhalf, thalf,
                 hoff, roff, toff, h0, r0, t0, h1, r1, t1,
                 scores_v, sem0, sem1):
    wid = lax.axis_index("s") * NC + lax.axis_index("c")
    iota = lax.iota(jnp.int32, LANES)
    base = wid * PER_W

    pltpu.sync_copy(hidx_hbm.at[pl.ds(base, PER_W)], hidx_v)
    pltpu.sync_copy(ridx_hbm.at[pl.ds(base, PER_W)], ridx_v)
    pltpu.sync_copy(tidx_hbm.at[pl.ds(base, PER_W)], tidx_v)

    def split_body(j, _):
        s = pl.ds(j * LANES, LANES)
        for idx_v, half_v, off_v in ((hidx_v, hhalf, hoff),
                                     (ridx_v, rhalf, roff),
                                     (tidx_v, thalf, toff)):
            v = idx_v[s]
            half_v[s] = lax.shift_left(lax.shift_right_logical(v, 15), 14) \
                + jnp.bitwise_and(v, 16383)
            off_v[s] = jnp.bitwise_and(lax.shift_right_logical(v, 14), 1)
        return 0

    lax.fori_loop(0, PER_W // LANES, split_body, 0)

    def fire(c, hb, rb, tb, sem):
        cc = jnp.minimum(c, NCH - 1)
        sl = pl.ds(cc * CHB, CHB)
        pltpu.async_copy(ents_hbm.at[hhalf.at[sl]], hb, sem)
        pltpu.async_copy(rels_hbm.at[rhalf.at[sl]], rb, sem)
        pltpu.async_copy(ents_hbm.at[thalf.at[sl]], tb, sem)

    def drain(hb, rb, tb, sem):
        pltpu.make_async_copy(ents_hbm.at[pl.ds(0, CHB)], hb, sem).wait()
        pltpu.make_async_copy(rels_hbm.at[pl.ds(0, CHB)], rb, sem).wait()
        pltpu.make_async_copy(ents_hbm.at[pl.ds(0, CHB)], tb, sem).wait()

    def compute(c, hb, rb, tb):
        def o_body(oi, _):
            s = pl.ds(c * CHB + oi * LANES, LANES)
            rows16 = oi * LANES + iota
            ho = hoff[s]
            ro = roff[s]
            to = toff[s]
            ssq = jnp.zeros((LANES,), jnp.float32)
            for d in range(DIM):
                vh = plsc.load_gather(hb, [rows16, ho + 2 * d])
                vr = plsc.load_gather(rb, [rows16, ro + 2 * d])
                vt = plsc.load_gather(tb, [rows16, to + 2 * d])
                diff = vh + vr - vt
                ssq = ssq + diff * diff
            scores_v[s] = ssq * _fast_rsqrt(jnp.maximum(ssq, TINY))
            return 0

        lax.fori_loop(0, CHB // LANES, o_body, 0)

    fire(jnp.int32(0), h0, r0, t0, sem0)

    def pair_body(i, _):
        c0 = i * 2
        drain(h0, r0, t0, sem0)
        fire(c0 + 1, h1, r1, t1, sem1)
        compute(c0, h0, r0, t0)
        drain(h1, r1, t1, sem1)
        fire(c0 + 2, h0, r0, t0, sem0)
        compute(c0 + 1, h1, r1, t1)
        return 0

    lax.fori_loop(0, NCH // 2, pair_body, 0)
    drain(h0, r0, t0, sem0)
    pltpu.sync_copy(scores_v, out_hbm.at[pl.ds(base, PER_W)])


@functools.partial(
    pl.kernel,
    out_type=jax.ShapeDtypeStruct((TOTAL,), jnp.float32),
    mesh=plsc.VectorSubcoreMesh(core_axis_name="c", subcore_axis_name="s"),
    compiler_params=pltpu.CompilerParams(
        needs_layout_passes=False, use_tc_tiling_on_sc=True),
    scratch_types=[
        pltpu.VMEM((PER_W,), jnp.int32),
        pltpu.VMEM((PER_W,), jnp.int32),
        pltpu.VMEM((PER_W,), jnp.int32),
        pltpu.VMEM((PER_W,), jnp.int32),
        pltpu.VMEM((PER_W,), jnp.int32),
        pltpu.VMEM((PER_W,), jnp.int32),
        pltpu.VMEM((PER_W,), jnp.int32),
        pltpu.VMEM((PER_W,), jnp.int32),
        pltpu.VMEM((PER_W,), jnp.int32),
        pltpu.VMEM((CHB, WIDE), jnp.float32),
        pltpu.VMEM((CHB, WIDE), jnp.float32),
        pltpu.VMEM((CHB, WIDE), jnp.float32),
        pltpu.VMEM((CHB, WIDE), jnp.float32),
        pltpu.VMEM((CHB, WIDE), jnp.float32),
        pltpu.VMEM((CHB, WIDE), jnp.float32),
        pltpu.VMEM((PER_W,), jnp.float32),
        pltpu.SemaphoreType.DMA,
        pltpu.SemaphoreType.DMA,
    ],
)
def _transe_sc(*refs):
    _transe_body(*refs)


TBLK = 32768
HALF_T = TBLK // 2


def _xpose_body(x_ref, y_ref):
    x = x_ref[...]
    norm = jnp.sqrt(jnp.sum(x * x, axis=0, keepdims=True))
    xn = x / jnp.maximum(norm, 1e-12)
    y_ref[...] = xn.reshape(WIDE, HALF_T).T


def _rows_table(table_t, nblk):
    return pl.pallas_call(
        _xpose_body,
        grid=(nblk,),
        in_specs=[pl.BlockSpec((DIM, TBLK), lambda j: (0, j))],
        out_specs=pl.BlockSpec((HALF_T, WIDE), lambda j: (j, 0)),
        out_shape=jax.ShapeDtypeStruct((nblk * HALF_T, WIDE), jnp.float32),
    )(table_t)


def kernel(ents_w, rels_w, heads, rels, tails, sources, heads_bad, rels_bad,
           tails_bad, sources_bad):
    b = heads.shape[0]
    n_ents = ents_w.shape[0]
    nblk = (n_ents + TBLK - 1) // TBLK
    ents_rows = _rows_table(ents_w.T, nblk)
    rels_rows = _rows_table(rels_w.T, 1)
    all_heads = jnp.concatenate([heads, heads_bad])
    all_rels = jnp.concatenate([rels, rels_bad])
    all_tails = jnp.concatenate([tails, tails_bad])
    out = _transe_sc(ents_rows, rels_rows, all_heads, all_rels, all_tails)
    return out[:b], out[b:]


# speedup vs baseline: 1.4712x; 1.0244x over previous
import functools

import jax
import jax.numpy as jnp
from jax import lax
from jax.experimental import pallas as pl
from jax.experimental.pallas import tpu as pltpu
from jax.experimental.pallas import tpu_sc as plsc

DIM = 64
WIDE = 128
LANES = 16
NC = 2
NS = 16
NW = NC * NS
TOTAL = 32768
PER_W = TOTAL // NW
CHB = 128
NCH = PER_W // CHB

EPS2 = 1e-24
TINY = 1e-35


def _fast_rsqrt(x):
    i = lax.bitcast_convert_type(x, jnp.int32)
    i = jnp.int32(0x5F3759DF) - lax.shift_right_arithmetic(i, 1)
    y = lax.bitcast_convert_type(i, jnp.float32)
    half_x = 0.5 * x
    for _ in range(3):
        y = y * (1.5 - half_x * y * y)
    return y


def _transe_body(ents_hbm, rels_hbm, hidx_hbm, ridx_hbm, tidx_hbm, out_hbm,
                 hidx_v, ridx_v, tidx_v, hhalf, rhalf, thalf,
                 hoff, roff, toff, h0, r0, t0, h1, r1, t1,
                 scores_v, sem0, sem1):
    wid = lax.axis_index("s") * NC + lax.axis_index("c")
    iota = lax.iota(jnp.int32, LANES)
    base = wid * PER_W

    pltpu.sync_copy(hidx_hbm.at[pl.ds(base, PER_W)], hidx_v)
    pltpu.sync_copy(ridx_hbm.at[pl.ds(base, PER_W)], ridx_v)
    pltpu.sync_copy(tidx_hbm.at[pl.ds(base, PER_W)], tidx_v)

    def split_body(j, _):
        s = pl.ds(j * LANES, LANES)
        for idx_v, half_v, off_v in ((hidx_v, hhalf, hoff),
                                     (tidx_v, thalf, toff)):
            v = idx_v[s]
            half_v[s] = lax.shift_left(lax.shift_right_logical(v, 15), 14) \
                + jnp.bitwise_and(v, 16383)
            off_v[s] = jnp.bitwise_and(lax.shift_right_logical(v, 14), 1)
        vr = ridx_v[s]
        rhalf[s] = vr
        roff[s] = jnp.bitwise_and(vr, 0)
        return 0

    lax.fori_loop(0, PER_W // LANES, split_body, 0)

    def fire(c, hb, rb, tb, sem):
        cc = jnp.minimum(c, NCH - 1)
        sl = pl.ds(cc * CHB, CHB)
        pltpu.async_copy(ents_hbm.at[hhalf.at[sl]], hb, sem)
        pltpu.async_copy(rels_hbm.at[rhalf.at[sl]], rb, sem)
        pltpu.async_copy(ents_hbm.at[thalf.at[sl]], tb, sem)

    def drain(hb, rb, tb, sem):
        pltpu.make_async_copy(ents_hbm.at[pl.ds(0, CHB)], hb, sem).wait()
        pltpu.make_async_copy(rels_hbm.at[pl.ds(0, CHB)], rb, sem).wait()
        pltpu.make_async_copy(ents_hbm.at[pl.ds(0, CHB)], tb, sem).wait()

    def compute(c, hb, rb, tb):
        def o_body(oi, _):
            s = pl.ds(c * CHB + oi * LANES, LANES)
            rows16 = oi * LANES + iota
            ho = hoff[s]
            ro = roff[s]
            to = toff[s]
            ssq = jnp.zeros((LANES,), jnp.float32)
            for d in range(DIM):
                vh = plsc.load_gather(hb, [rows16, ho + 2 * d])
                vr = plsc.load_gather(rb, [rows16, ro + 2 * d])
                vt = plsc.load_gather(tb, [rows16, to + 2 * d])
                diff = vh + vr - vt
                ssq = ssq + diff * diff
            scores_v[s] = ssq * _fast_rsqrt(jnp.maximum(ssq, TINY))
            return 0

        lax.fori_loop(0, CHB // LANES, o_body, 0)

    fire(jnp.int32(0), h0, r0, t0, sem0)

    def pair_body(i, _):
        c0 = i * 2
        drain(h0, r0, t0, sem0)
        fire(c0 + 1, h1, r1, t1, sem1)
        compute(c0, h0, r0, t0)
        drain(h1, r1, t1, sem1)
        fire(c0 + 2, h0, r0, t0, sem0)
        compute(c0 + 1, h1, r1, t1)
        return 0

    lax.fori_loop(0, NCH // 2, pair_body, 0)
    drain(h0, r0, t0, sem0)
    pltpu.sync_copy(scores_v, out_hbm.at[pl.ds(base, PER_W)])


@functools.partial(
    pl.kernel,
    out_type=jax.ShapeDtypeStruct((TOTAL,), jnp.float32),
    mesh=plsc.VectorSubcoreMesh(core_axis_name="c", subcore_axis_name="s"),
    compiler_params=pltpu.CompilerParams(
        needs_layout_passes=False, use_tc_tiling_on_sc=True),
    scratch_types=[
        pltpu.VMEM((PER_W,), jnp.int32),
        pltpu.VMEM((PER_W,), jnp.int32),
        pltpu.VMEM((PER_W,), jnp.int32),
        pltpu.VMEM((PER_W,), jnp.int32),
        pltpu.VMEM((PER_W,), jnp.int32),
        pltpu.VMEM((PER_W,), jnp.int32),
        pltpu.VMEM((PER_W,), jnp.int32),
        pltpu.VMEM((PER_W,), jnp.int32),
        pltpu.VMEM((PER_W,), jnp.int32),
        pltpu.VMEM((CHB, WIDE), jnp.float32),
        pltpu.VMEM((CHB, WIDE), jnp.float32),
        pltpu.VMEM((CHB, WIDE), jnp.float32),
        pltpu.VMEM((CHB, WIDE), jnp.float32),
        pltpu.VMEM((CHB, WIDE), jnp.float32),
        pltpu.VMEM((CHB, WIDE), jnp.float32),
        pltpu.VMEM((PER_W,), jnp.float32),
        pltpu.SemaphoreType.DMA,
        pltpu.SemaphoreType.DMA,
    ],
)
def _transe_sc(*refs):
    _transe_body(*refs)


TBLK = 32768
TBLK_R = 2048


def _make_xpose_body(tblk):
    def _xpose_body(x_ref, y_ref):
        x = x_ref[...]
        norm = jnp.sqrt(jnp.sum(x * x, axis=0, keepdims=True))
        xn = x / jnp.maximum(norm, 1e-12)
        y_ref[...] = xn.reshape(WIDE, tblk // 2).T
    return _xpose_body


def _rows_table(table_t, nblk, tblk):
    half = tblk // 2
    return pl.pallas_call(
        _make_xpose_body(tblk),
        grid=(nblk,),
        in_specs=[pl.BlockSpec((DIM, tblk), lambda j: (0, j))],
        out_specs=pl.BlockSpec((half, WIDE), lambda j: (j, 0)),
        out_shape=jax.ShapeDtypeStruct((nblk * half, WIDE), jnp.float32),
    )(table_t)


def kernel(ents_w, rels_w, heads, rels, tails, sources, heads_bad, rels_bad,
           tails_bad, sources_bad):
    b = heads.shape[0]
    n_ents = ents_w.shape[0]
    nblk = (n_ents + TBLK - 1) // TBLK
    ents_rows = _rows_table(ents_w.T, nblk, TBLK)
    rels_rows = _rows_table(rels_w.T, 1, TBLK_R)
    all_heads = jnp.concatenate([heads, heads_bad])
    all_rels = jnp.concatenate([rels, rels_bad])
    all_tails = jnp.concatenate([tails, tails_bad])
    out = _transe_sc(ents_rows, rels_rows, all_heads, all_rels, all_tails)
    return out[:b], out[b:]
